# baseline (speedup 1.0000x reference)
import functools

import jax
import jax.numpy as jnp
from jax import lax
from jax.experimental import pallas as pl
from jax.experimental.pallas import tpu as pltpu
from jax.experimental.pallas import tpu_sc as plsc

_BATCH = 16384
_DIM = 32
_NC = 2
_NS = 16
_NW = _NC * _NS
_BPW = _BATCH // _NW

_mesh = plsc.VectorSubcoreMesh(core_axis_name="c", subcore_axis_name="s")


def _bpr_body(user_id, pos_id, neg_id, user_table, item_table,
            pos_hbm, neg_hbm,
            u_idx, p_idx, n_idx, u_rows, p_rows, n_rows,
            pos_v, neg_v, sbuf, tbuf, sem):
    wid = lax.axis_index("s") * _NC + lax.axis_index("c")
    base = wid * _BPW

    pltpu.sync_copy(user_id.at[pl.ds(base, _BPW)], u_idx)
    pltpu.sync_copy(pos_id.at[pl.ds(base, _BPW)], p_idx)
    pltpu.sync_copy(neg_id.at[pl.ds(base, _BPW)], n_idx)

    cu = pltpu.async_copy(user_table.at[u_idx], u_rows, sem)
    cp = pltpu.async_copy(item_table.at[p_idx], p_rows, sem)
    cn = pltpu.async_copy(item_table.at[n_idx], n_rows, sem)
    cu.wait()
    cp.wait()
    cn.wait()

    lane = lax.iota(jnp.int32, 16)

    def body(g, carry):
        gbase = pl.multiple_of(g * 16, 16)
        for r in range(16):
            i = gbase + r
            u0 = u_rows[i, pl.ds(0, 16)]
            u1 = u_rows[i, pl.ds(16, 16)]
            p0 = p_rows[i, pl.ds(0, 16)]
            p1 = p_rows[i, pl.ds(16, 16)]
            n0 = n_rows[i, pl.ds(0, 16)]
            n1 = n_rows[i, pl.ds(16, 16)]
            sbuf[r, pl.ds(0, 16)] = u0 * p0 + u1 * p1
            tbuf[r, pl.ds(0, 16)] = u0 * n0 + u1 * n1
        accp = jnp.zeros((16,), jnp.float32)
        accn = jnp.zeros((16,), jnp.float32)
        for j in range(16):
            col = jnp.full((16,), j, jnp.int32)
            accp = accp + plsc.load_gather(sbuf, [lane, col])
            accn = accn + plsc.load_gather(tbuf, [lane, col])
        pos_v[pl.ds(gbase, 16)] = accp
        neg_v[pl.ds(gbase, 16)] = accn
        return carry

    lax.fori_loop(0, _BPW // 16, body, 0)

    pltpu.sync_copy(pos_v, pos_hbm.at[pl.ds(base, _BPW)])
    pltpu.sync_copy(neg_v, neg_hbm.at[pl.ds(base, _BPW)])


def _build(interpret=False):
    return pl.kernel(
        _bpr_body,
        out_type=(
            jax.ShapeDtypeStruct((_BATCH,), jnp.float32),
            jax.ShapeDtypeStruct((_BATCH,), jnp.float32),
        ),
        mesh=_mesh,
        compiler_params=pltpu.CompilerParams(
            needs_layout_passes=False, use_tc_tiling_on_sc=False),
        scratch_types=[
            pltpu.VMEM((_BPW,), jnp.int32),
            pltpu.VMEM((_BPW,), jnp.int32),
            pltpu.VMEM((_BPW,), jnp.int32),
            pltpu.VMEM((_BPW, _DIM), jnp.float32),
            pltpu.VMEM((_BPW, _DIM), jnp.float32),
            pltpu.VMEM((_BPW, _DIM), jnp.float32),
            pltpu.VMEM((_BPW,), jnp.float32),
            pltpu.VMEM((_BPW,), jnp.float32),
            pltpu.VMEM((16, 17), jnp.float32),
            pltpu.VMEM((16, 17), jnp.float32),
            pltpu.SemaphoreType.DMA,
        ],
        interpret=interpret,
    )


_bpr_sc = _build()


def kernel(user_id, pos_id, neg_id, user_table, item_table):
    return _bpr_sc(user_id, pos_id, neg_id, user_table, item_table)


# speedup vs baseline: 2.7435x; 2.7435x over previous
import functools

import jax
import jax.numpy as jnp
from jax import lax
from jax.experimental import pallas as pl
from jax.experimental.pallas import tpu as pltpu
from jax.experimental.pallas import tpu_sc as plsc

_BATCH = 16384
_DIM = 32
_NC = 2
_NS = 16
_NW = _NC * _NS
_BPW = _BATCH // _NW
_G = 8
_NGRP = _BPW // _G

_mesh = plsc.VectorSubcoreMesh(core_axis_name="c", subcore_axis_name="s")


def _bpr_body(user_id, pos_id, neg_id, ut, it,
              pos_hbm, neg_hbm,
              u_idx, p_idx, n_idx,
              ubuf, pbuf, nbuf,
              pos_v, neg_v, sem):
    wid = lax.axis_index("s") * _NC + lax.axis_index("c")
    base = wid * _BPW

    pltpu.sync_copy(user_id.at[pl.ds(base, _BPW)], u_idx)
    pltpu.sync_copy(pos_id.at[pl.ds(base, _BPW)], p_idx)
    pltpu.sync_copy(neg_id.at[pl.ds(base, _BPW)], n_idx)
    lane = lax.iota(jnp.int32, 16)

    def body(g, carry):
        gbase = pl.multiple_of(g * 16, 16)
        iv_u = u_idx[pl.ds(gbase, 16)]
        iv_p = p_idx[pl.ds(gbase, 16)]
        iv_n = n_idx[pl.ds(gbase, 16)]
        lid_u = iv_u & 127
        lid_p = iv_p & 127
        lid_n = iv_n & 127
        halves = []
        for h in range(2):
            copies = []
            for tab, buf, iv in ((ut, ubuf, iv_u), (it, pbuf, iv_p),
                                 (it, nbuf, iv_n)):
                for j in range(_G):
                    idv = iv[h * _G + j]
                    blk = pl.multiple_of(idv & -128, 128)
                    copies.append(
                        pltpu.async_copy(tab.at[:, pl.ds(blk, 128)],
                                         buf.at[j], sem))
            for cp in copies:
                cp.wait()
            jvec = jnp.clip(lane - h * _G, 0, _G - 1)
            accp = jnp.zeros((16,), jnp.float32)
            accn = jnp.zeros((16,), jnp.float32)
            for d in range(_DIM):
                dcol = jnp.full((16,), d, jnp.int32)
                du = plsc.load_gather(ubuf, [jvec, dcol, lid_u])
                dp = plsc.load_gather(pbuf, [jvec, dcol, lid_p])
                dn = plsc.load_gather(nbuf, [jvec, dcol, lid_n])
                accp = accp + du * dp
                accn = accn + du * dn
            halves.append((accp, accn))
        lo = lane < _G
        pos_v[pl.ds(gbase, 16)] = jnp.where(lo, halves[0][0], halves[1][0])
        neg_v[pl.ds(gbase, 16)] = jnp.where(lo, halves[0][1], halves[1][1])
        return carry

    lax.fori_loop(0, _BPW // 16, body, 0)

    pltpu.sync_copy(pos_v, pos_hbm.at[pl.ds(base, _BPW)])
    pltpu.sync_copy(neg_v, neg_hbm.at[pl.ds(base, _BPW)])


def _build(interpret=False):
    return pl.kernel(
        _bpr_body,
        out_type=(
            jax.ShapeDtypeStruct((_BATCH,), jnp.float32),
            jax.ShapeDtypeStruct((_BATCH,), jnp.float32),
        ),
        mesh=_mesh,
        compiler_params=pltpu.CompilerParams(needs_layout_passes=False),
        scratch_types=[
            pltpu.VMEM((_BPW,), jnp.int32),
            pltpu.VMEM((_BPW,), jnp.int32),
            pltpu.VMEM((_BPW,), jnp.int32),
            pltpu.VMEM((_G, _DIM, 128), jnp.float32),
            pltpu.VMEM((_G, _DIM, 128), jnp.float32),
            pltpu.VMEM((_G, _DIM, 128), jnp.float32),
            pltpu.VMEM((_BPW,), jnp.float32),
            pltpu.VMEM((_BPW,), jnp.float32),
            pltpu.SemaphoreType.DMA,
        ],
        interpret=interpret,
    )


_bpr_sc = _build()


def kernel(user_id, pos_id, neg_id, user_table, item_table):
    return _bpr_sc(user_id, pos_id, neg_id, user_table.T, item_table.T)
